# baseline (speedup 1.0000x reference)
import functools

import jax
import jax.numpy as jnp
from jax import lax
from jax.experimental import pallas as pl
from jax.experimental.pallas import tpu as pltpu
from jax.experimental.pallas import tpu_sc as plsc

B = 2
C = 128
N = 10000
K = 16
OUT_C = 128
GROUPS = 4
BN = B * N

NC = 2
NS = 16
NW = NC * NS
CH = 8
ROWS = CH * K
BCHUNKS = N // CH
BASE_CHUNKS = BCHUNKS // NW
XTRA = BCHUNKS - BASE_CHUNKS * NW
MAXN = (BASE_CHUNKS + 1) * CH
L = 16


def _sc_body(xt_hbm, i0_hbm, i1_hbm, out_hbm,
             i0v, i1v, r0, r1, ov,
             gsem0, gsem1, osem0, osem1):
    gsems = (gsem0, gsem1)
    osems = (osem0, osem1)
    wid = lax.axis_index("s") * NC + lax.axis_index("c")
    start = wid * (BASE_CHUNKS * CH) + CH * jnp.minimum(wid, XTRA)
    nchunk = jnp.where(wid < XTRA, BASE_CHUNKS + 1, BASE_CHUNKS)
    obase = start * C
    ibase = start * K

    @pl.when(wid < XTRA)
    def _():
        pltpu.sync_copy(i0_hbm.at[pl.ds(ibase, MAXN * K)], i0v)
        pltpu.sync_copy(i1_hbm.at[pl.ds(ibase, MAXN * K)], i1v)

    @pl.when(wid >= XTRA)
    def _():
        nk = BASE_CHUNKS * CH * K
        pltpu.sync_copy(i0_hbm.at[pl.ds(ibase, nk)], i0v.at[pl.ds(0, nk)])
        pltpu.sync_copy(i1_hbm.at[pl.ds(ibase, nk)], i1v.at[pl.ds(0, nk)])

    def gather_descs(c, s):
        off = c * ROWS
        d0 = pltpu.make_async_copy(
            xt_hbm.at[i0v.at[pl.ds(off, ROWS)]], r0.at[s], gsems[s])
        d1 = pltpu.make_async_copy(
            xt_hbm.at[i1v.at[pl.ds(off, ROWS)]], r1.at[s], gsems[s])
        return d0, d1

    def gather_start(c, s):
        d0, d1 = gather_descs(c, s)
        d0.start()
        d1.start()

    def gather_wait(c, s):
        d0, d1 = gather_descs(c, s)
        d0.wait()
        d1.wait()

    def store_desc(c, s):
        return pltpu.make_async_copy(
            ov.at[s], out_hbm.at[pl.ds(obase + c * (CH * C), CH * C)],
            osems[s])

    def compute(c, s):
        @pl.loop(0, CH)
        def _(n):
            row = n * K
            for g in range(C // L):
                sl = pl.ds(g * L, L)
                a = r0[s, row, sl] - r1[s, row, sl]
                for kk in range(1, K):
                    a = jnp.maximum(a, r0[s, row + kk, sl] - r1[s, row + kk, sl])
                ov[s, pl.ds(n * C + g * L, L)] = a

    gather_start(0, 0)
    gather_start(1, 1)

    odd = lax.rem(nchunk, 2)
    twon = nchunk - odd

    @pl.loop(0, twon, step=2)
    def _(c0):
        for s in range(2):
            c = c0 + s
            gather_wait(c, s)

            @pl.when(c >= 2)
            def _():
                store_desc(c - 2, s).wait()

            compute(c, s)
            store_desc(c, s).start()

            @pl.when(c + 2 < nchunk)
            def _():
                gather_start(c + 2, s)

    @pl.when(odd == 1)
    def _():
        gather_wait(twon, 0)
        store_desc(twon - 2, 0).wait()
        compute(twon, 0)
        store_desc(twon, 0).start()
        store_desc(twon - 1, 1).wait()
        store_desc(twon, 0).wait()

    @pl.when(odd == 0)
    def _():
        store_desc(twon - 2, 0).wait()
        store_desc(twon - 1, 1).wait()


def _sc_maxrel(xt, i0, i1):
    mesh = plsc.VectorSubcoreMesh(core_axis_name="c", subcore_axis_name="s")
    kfn = functools.partial(
        pl.kernel,
        mesh=mesh,
        out_type=jax.ShapeDtypeStruct((N * C,), jnp.float32),
        scratch_types=[
            pltpu.VMEM((MAXN * K,), jnp.int32),
            pltpu.VMEM((MAXN * K,), jnp.int32),
            pltpu.VMEM((2, ROWS, C), jnp.float32),
            pltpu.VMEM((2, ROWS, C), jnp.float32),
            pltpu.VMEM((2, CH * C), jnp.float32),
            pltpu.SemaphoreType.DMA,
            pltpu.SemaphoreType.DMA,
            pltpu.SemaphoreType.DMA,
            pltpu.SemaphoreType.DMA,
        ],
    )(_sc_body)
    return kfn(xt, i0, i1)


def _conv_body(x_ref, xj_ref, ax_ref, aj_ref, b_ref, o_ref):
    xb = x_ref[...]
    xjb = xj_ref[...]
    acc = lax.dot_general(ax_ref[...], xb, (((1,), (0,)), ((), ())),
                          preferred_element_type=jnp.float32)
    acc = acc + lax.dot_general(aj_ref[...], xjb, (((1,), (1,)), ((), ())),
                                preferred_element_type=jnp.float32)
    o_ref[...] = jnp.maximum(acc + b_ref[...], 0.0)


def _conv(xc, xj_nc, ax, aj, b2):
    nt = 2048
    grid = (pl.cdiv(N, nt),)
    return pl.pallas_call(
        _conv_body,
        grid=grid,
        in_specs=[
            pl.BlockSpec((C, nt), lambda t: (0, t)),
            pl.BlockSpec((nt, C), lambda t: (t, 0)),
            pl.BlockSpec((OUT_C, C), lambda t: (0, 0)),
            pl.BlockSpec((OUT_C, C), lambda t: (0, 0)),
            pl.BlockSpec((OUT_C, 1), lambda t: (0, 0)),
        ],
        out_specs=pl.BlockSpec((OUT_C, nt), lambda t: (0, t)),
        out_shape=jax.ShapeDtypeStruct((OUT_C, N), jnp.float32),
    )(xc, xj_nc, ax, aj, b2)


def kernel(x, edge_index, W, b):
    xsq = x[:, :, :, 0]
    xt = jnp.transpose(xsq, (0, 2, 1))

    wr = W.reshape(GROUPS, OUT_C // GROUPS, C // GROUPS, 2)
    ax = jax.scipy.linalg.block_diag(*[wr[g, :, :, 0] for g in range(GROUPS)])
    aj = jax.scipy.linalg.block_diag(*[wr[g, :, :, 1] for g in range(GROUPS)])
    b2 = b.reshape(OUT_C, 1)

    outs = []
    for bb in range(B):
        i0 = edge_index[0, bb].reshape(N * K)
        i1 = edge_index[1, bb].reshape(N * K)
        xj = _sc_maxrel(xt[bb], i0, i1).reshape(N, C)
        outs.append(_conv(xsq[bb], xj, ax, aj, b2))
    return jnp.stack(outs)[..., None]


# speedup vs baseline: 1.0845x; 1.0845x over previous
import functools

import jax
import jax.numpy as jnp
from jax import lax
from jax.experimental import pallas as pl
from jax.experimental.pallas import tpu as pltpu
from jax.experimental.pallas import tpu_sc as plsc

B = 2
C = 128
N = 10000
K = 16
OUT_C = 128
GROUPS = 4
BN = B * N

NC = 2
NS = 16
NW = NC * NS
CH = 8
ROWS = CH * K
BCHUNKS = BN // CH
BASE_CHUNKS = BCHUNKS // NW
XTRA = BCHUNKS - BASE_CHUNKS * NW
MAXN = (BASE_CHUNKS + 1) * CH
L = 16


def _sc_body(xt_hbm, i0_hbm, i1_hbm, out_hbm,
             i0v, i1v, r0, r1, ov,
             gsem0, gsem1, osem0, osem1):
    gsems = (gsem0, gsem1)
    osems = (osem0, osem1)
    wid = lax.axis_index("s") * NC + lax.axis_index("c")
    start = wid * (BASE_CHUNKS * CH) + CH * jnp.minimum(wid, XTRA)
    nchunk = jnp.where(wid < XTRA, BASE_CHUNKS + 1, BASE_CHUNKS)
    obase = start * C
    ibase = start * K

    @pl.when(wid < XTRA)
    def _():
        pltpu.sync_copy(i0_hbm.at[pl.ds(ibase, MAXN * K)], i0v)
        pltpu.sync_copy(i1_hbm.at[pl.ds(ibase, MAXN * K)], i1v)

    @pl.when(wid >= XTRA)
    def _():
        nk = BASE_CHUNKS * CH * K
        pltpu.sync_copy(i0_hbm.at[pl.ds(ibase, nk)], i0v.at[pl.ds(0, nk)])
        pltpu.sync_copy(i1_hbm.at[pl.ds(ibase, nk)], i1v.at[pl.ds(0, nk)])

    def gather_descs(c, s):
        off = c * ROWS
        d0 = pltpu.make_async_copy(
            xt_hbm.at[i0v.at[pl.ds(off, ROWS)]], r0.at[s], gsems[s])
        d1 = pltpu.make_async_copy(
            xt_hbm.at[i1v.at[pl.ds(off, ROWS)]], r1.at[s], gsems[s])
        return d0, d1

    def gather_start(c, s):
        d0, d1 = gather_descs(c, s)
        d0.start()
        d1.start()

    def gather_wait(c, s):
        d0, d1 = gather_descs(c, s)
        d0.wait()
        d1.wait()

    def store_desc(c, s):
        return pltpu.make_async_copy(
            ov.at[s], out_hbm.at[pl.ds(obase + c * (CH * C), CH * C)],
            osems[s])

    def compute(c, s):
        @pl.loop(0, CH)
        def _(n):
            row = n * K
            for g in range(C // L):
                sl = pl.ds(g * L, L)
                a = r0[s, row, sl] - r1[s, row, sl]
                for kk in range(1, K):
                    a = jnp.maximum(a, r0[s, row + kk, sl] - r1[s, row + kk, sl])
                ov[s, pl.ds(n * C + g * L, L)] = a

    gather_start(0, 0)
    gather_start(1, 1)

    odd = lax.rem(nchunk, 2)
    twon = nchunk - odd

    @pl.loop(0, twon, step=2)
    def _(c0):
        for s in range(2):
            c = c0 + s
            gather_wait(c, s)

            @pl.when(c >= 2)
            def _():
                store_desc(c - 2, s).wait()

            compute(c, s)
            store_desc(c, s).start()

            @pl.when(c + 2 < nchunk)
            def _():
                gather_start(c + 2, s)

    @pl.when(odd == 1)
    def _():
        gather_wait(twon, 0)
        store_desc(twon - 2, 0).wait()
        compute(twon, 0)
        store_desc(twon, 0).start()
        store_desc(twon - 1, 1).wait()
        store_desc(twon, 0).wait()

    @pl.when(odd == 0)
    def _():
        store_desc(twon - 2, 0).wait()
        store_desc(twon - 1, 1).wait()


def _sc_maxrel(xt, i0, i1):
    mesh = plsc.VectorSubcoreMesh(core_axis_name="c", subcore_axis_name="s")
    kfn = functools.partial(
        pl.kernel,
        mesh=mesh,
        out_type=jax.ShapeDtypeStruct((BN * C,), jnp.float32),
        scratch_types=[
            pltpu.VMEM((MAXN * K,), jnp.int32),
            pltpu.VMEM((MAXN * K,), jnp.int32),
            pltpu.VMEM((2, ROWS, C), jnp.float32),
            pltpu.VMEM((2, ROWS, C), jnp.float32),
            pltpu.VMEM((2, CH * C), jnp.float32),
            pltpu.SemaphoreType.DMA,
            pltpu.SemaphoreType.DMA,
            pltpu.SemaphoreType.DMA,
            pltpu.SemaphoreType.DMA,
        ],
    )(_sc_body)
    return kfn(xt, i0, i1)


def _conv_body(x_ref, xj_ref, ax_ref, aj_ref, b_ref, o_ref):
    xb = x_ref[0]
    xjb = xj_ref[0]
    acc = lax.dot_general(ax_ref[...], xb, (((1,), (0,)), ((), ())),
                          preferred_element_type=jnp.float32)
    acc = acc + lax.dot_general(aj_ref[...], xjb, (((1,), (1,)), ((), ())),
                                preferred_element_type=jnp.float32)
    o_ref[0] = jnp.maximum(acc + b_ref[...], 0.0)


def _conv(xcn, xj_nc, ax, aj, b2):
    nt = 2048
    grid = (B, pl.cdiv(N, nt))
    return pl.pallas_call(
        _conv_body,
        grid=grid,
        in_specs=[
            pl.BlockSpec((1, C, nt), lambda bb, t: (bb, 0, t)),
            pl.BlockSpec((1, nt, C), lambda bb, t: (bb, t, 0)),
            pl.BlockSpec((OUT_C, C), lambda bb, t: (0, 0)),
            pl.BlockSpec((OUT_C, C), lambda bb, t: (0, 0)),
            pl.BlockSpec((OUT_C, 1), lambda bb, t: (0, 0)),
        ],
        out_specs=pl.BlockSpec((1, OUT_C, nt), lambda bb, t: (bb, 0, t)),
        out_shape=jax.ShapeDtypeStruct((B, OUT_C, N), jnp.float32),
    )(xcn, xj_nc, ax, aj, b2)


def kernel(x, edge_index, W, b):
    xsq = x[:, :, :, 0]
    xt = jnp.transpose(xsq, (0, 2, 1)).reshape(BN, C)
    offs = (jnp.arange(B, dtype=jnp.int32) * N).reshape(1, B, 1, 1)
    ef = edge_index + offs
    i0 = ef[0].reshape(BN * K)
    i1 = ef[1].reshape(BN * K)

    xj = _sc_maxrel(xt, i0, i1).reshape(B, N, C)

    wr = W.reshape(GROUPS, OUT_C // GROUPS, C // GROUPS, 2)
    ax = jax.scipy.linalg.block_diag(*[wr[g, :, :, 0] for g in range(GROUPS)])
    aj = jax.scipy.linalg.block_diag(*[wr[g, :, :, 1] for g in range(GROUPS)])

    out = _conv(xsq, xj, ax, aj, b.reshape(OUT_C, 1))
    return out[..., None]
